# baseline (speedup 1.0000x reference)
import functools

import jax
import jax.numpy as jnp
from jax import lax
from jax.experimental import pallas as pl
from jax.experimental.pallas import tpu as pltpu
from jax.experimental.pallas import tpu_sc as plsc

_NTOK = 8192
_NW = 32
_BPW = _NTOK // _NW
_CUT = (0, 19997, 39997, 199997, 1000000)
_DS = (1024, 256, 64, 16)
_GW = (1024, 256, 128, 128)
_CHUNKS = (32, 128, 128, 128)
_DPROJ = 1024
_SCALE = float(_DPROJ) ** 0.5


def _sc_gather(inp_flat, emb0, emb1, emb2g, emb3g):
    mesh = plsc.VectorSubcoreMesh(core_axis_name="c", subcore_axis_name="s")
    out_type = tuple(
        jax.ShapeDtypeStruct((_NTOK, w), jnp.float32) for w in _GW
    )
    scratch = [
        pltpu.VMEM((_BPW,), jnp.int32),
        pltpu.VMEM((_BPW,), jnp.int32),
        pltpu.VMEM((_BPW,), jnp.int32),
        pltpu.VMEM((_BPW,), jnp.int32),
        pltpu.VMEM((_BPW,), jnp.int32),
        pltpu.VMEM((_CHUNKS[0], _GW[0]), jnp.float32),
        pltpu.VMEM((_CHUNKS[1], _GW[1]), jnp.float32),
        pltpu.VMEM((_CHUNKS[2], _GW[2]), jnp.float32),
        pltpu.VMEM((_CHUNKS[3], _GW[3]), jnp.float32),
        pltpu.SemaphoreType.DMA,
    ]

    @functools.partial(pl.kernel, mesh=mesh, out_type=out_type,
                       scratch_types=scratch)
    def body(inp_h, e0_h, e1_h, e2_h, e3_h, g0_h, g1_h, g2_h, g3_h,
             tok_v, i0_v, i1_v, i2_v, i3_v, b0_v, b1_v, b2_v, b3_v, sem):
        wid = lax.axis_index("s") * 2 + lax.axis_index("c")
        base = wid * _BPW
        pltpu.sync_copy(inp_h.at[pl.ds(base, _BPW)], tok_v)
        for i in range(_BPW // 16):
            sl = pl.ds(i * 16, 16)
            t = tok_v[sl]
            i0_v[sl] = jnp.clip(t - _CUT[0], 0, _CUT[1] - _CUT[0] - 1)
            i1_v[sl] = jnp.clip(t - _CUT[1], 0, _CUT[2] - _CUT[1] - 1)
            i2_v[sl] = jnp.clip(t - _CUT[2], 0, _CUT[3] - _CUT[2] - 1) >> 1
            i3_v[sl] = jnp.clip(t - _CUT[3], 0, _CUT[4] - _CUT[3] - 1) >> 3
        tabs = (e0_h, e1_h, e2_h, e3_h)
        outs = (g0_h, g1_h, g2_h, g3_h)
        bufs = (b0_v, b1_v, b2_v, b3_v)
        idxs = (i0_v, i1_v, i2_v, i3_v)
        for c in range(4):
            ch = _CHUNKS[c]
            for j in range(_BPW // ch):
                isl = pl.ds(j * ch, ch)
                pltpu.async_copy(
                    tabs[c].at[idxs[c].at[isl]], bufs[c], sem).wait()
                pltpu.sync_copy(bufs[c], outs[c].at[pl.ds(base + j * ch, ch)])

    return body(inp_flat, emb0, emb1, emb2g, emb3g)


def _tc_project(inp3, g0, g1, g2, g3, p0, p1, p2, p3):
    blk = 256
    nb = _NTOK // blk

    def body(i_ref, g0_ref, g1_ref, g2_ref, g3_ref,
             p0_ref, p1_ref, p2_ref, p3_ref, o_ref):
        tok = i_ref[0]

        m0 = (tok >= _CUT[0]) & (tok < _CUT[1])
        x0 = jnp.where(m0, g0_ref[...], 0.0).astype(jnp.bfloat16)

        m1 = (tok >= _CUT[1]) & (tok < _CUT[2])
        x1 = jnp.where(m1, g1_ref[...], 0.0).astype(jnp.bfloat16)

        m2 = (tok >= _CUT[2]) & (tok < _CUT[3])
        l2 = jnp.clip(tok - _CUT[2], 0, _CUT[3] - _CUT[2] - 1)
        half = (l2 & 1) == 1
        gr2 = g2_ref[...]
        x2f = jnp.where(half, gr2[:, 64:], gr2[:, :64])
        x2 = jnp.where(m2, x2f, 0.0).astype(jnp.bfloat16)

        m3 = (tok >= _CUT[3]) & (tok < _CUT[4])
        l3 = jnp.clip(tok - _CUT[3], 0, _CUT[4] - _CUT[3] - 1)
        slot = l3 & 7
        gr3 = g3_ref[...]
        x3f = jnp.zeros((blk, 16), jnp.float32)
        for s in range(8):
            x3f = jnp.where(slot == s,
                            gr3[:, s * 16:(s + 1) * 16], x3f)
        x3 = jnp.where(m3, x3f, 0.0).astype(jnp.bfloat16)

        dn = (((1,), (1,)), ((), ()))
        acc = lax.dot_general(x0, p0_ref[...], dn,
                              preferred_element_type=jnp.float32)
        acc += lax.dot_general(x1, p1_ref[...], dn,
                               preferred_element_type=jnp.float32)
        acc += lax.dot_general(x2, p2_ref[...], dn,
                               preferred_element_type=jnp.float32)
        acc += lax.dot_general(x3, p3_ref[...], dn,
                               preferred_element_type=jnp.float32)
        o_ref[...] = acc * _SCALE

    in_specs = [
        pl.BlockSpec((1, blk, 1), lambda i: (i, 0, 0)),
        pl.BlockSpec((blk, _GW[0]), lambda i: (i, 0)),
        pl.BlockSpec((blk, _GW[1]), lambda i: (i, 0)),
        pl.BlockSpec((blk, _GW[2]), lambda i: (i, 0)),
        pl.BlockSpec((blk, _GW[3]), lambda i: (i, 0)),
        pl.BlockSpec((_DPROJ, _DS[0]), lambda i: (0, 0)),
        pl.BlockSpec((_DPROJ, _DS[1]), lambda i: (0, 0)),
        pl.BlockSpec((_DPROJ, _DS[2]), lambda i: (0, 0)),
        pl.BlockSpec((_DPROJ, _DS[3]), lambda i: (0, 0)),
    ]
    return pl.pallas_call(
        body,
        grid=(nb,),
        in_specs=in_specs,
        out_specs=pl.BlockSpec((blk, _DPROJ), lambda i: (i, 0)),
        out_shape=jax.ShapeDtypeStruct((_NTOK, _DPROJ), jnp.float32),
        compiler_params=pltpu.CompilerParams(
            dimension_semantics=("arbitrary",)),
    )(inp3, g0, g1, g2, g3, p0, p1, p2, p3)


def kernel(inp, emb0, emb1, emb2, emb3, proj0, proj1, proj2, proj3):
    inp_flat = inp.reshape(-1)
    emb2g = emb2.reshape(-1, 128)
    emb3p = jnp.concatenate(
        [emb3, jnp.zeros((5, 16), jnp.float32)], axis=0)
    emb3g = emb3p.reshape(-1, 128)
    g0, g1, g2, g3 = _sc_gather(inp_flat, emb0, emb1, emb2g, emb3g)
    inp3 = inp_flat.reshape(_NTOK // 256, 256, 1)
    pbf = [p.astype(jnp.bfloat16) for p in (proj0, proj1, proj2, proj3)]
    out = _tc_project(inp3, g0, g1, g2, g3, *pbf)
    return out.reshape(inp.shape + (_DPROJ,))


# speedup vs baseline: 1.0008x; 1.0008x over previous
import functools

import jax
import jax.numpy as jnp
from jax import lax
from jax.experimental import pallas as pl
from jax.experimental.pallas import tpu as pltpu
from jax.experimental.pallas import tpu_sc as plsc

_NTOK = 8192
_NW = 32
_BPW = _NTOK // _NW
_CUT = (0, 19997, 39997, 199997, 1000000)
_DS = (1024, 256, 64, 16)
_GW = (1024, 256, 128, 128)
_CHUNKS = (32, 128, 128, 128)
_DPROJ = 1024
_SCALE = float(_DPROJ) ** 0.5


def _sc_gather(inp_flat, emb0, emb1, emb2g, emb3g):
    mesh = plsc.VectorSubcoreMesh(core_axis_name="c", subcore_axis_name="s")
    out_type = tuple(
        jax.ShapeDtypeStruct((_NTOK, w), jnp.float32) for w in _GW
    )
    scratch = [
        pltpu.VMEM((_BPW,), jnp.int32),
        pltpu.VMEM((_BPW,), jnp.int32),
        pltpu.VMEM((_BPW,), jnp.int32),
        pltpu.VMEM((_BPW,), jnp.int32),
        pltpu.VMEM((_BPW,), jnp.int32),
        pltpu.VMEM((_CHUNKS[0], _GW[0]), jnp.float32),
        pltpu.VMEM((_CHUNKS[1], _GW[1]), jnp.float32),
        pltpu.VMEM((_CHUNKS[2], _GW[2]), jnp.float32),
        pltpu.VMEM((_CHUNKS[3], _GW[3]), jnp.float32),
        pltpu.SemaphoreType.DMA,
    ]

    @functools.partial(pl.kernel, mesh=mesh, out_type=out_type,
                       scratch_types=scratch)
    def body(inp_h, e0_h, e1_h, e2_h, e3_h, g0_h, g1_h, g2_h, g3_h,
             tok_v, i0_v, i1_v, i2_v, i3_v, b0_v, b1_v, b2_v, b3_v, sem):
        wid = lax.axis_index("s") * 2 + lax.axis_index("c")
        base = wid * _BPW
        with jax.named_scope("sc_idx"):
            pltpu.sync_copy(inp_h.at[pl.ds(base, _BPW)], tok_v)
            for i in range(_BPW // 16):
                sl = pl.ds(i * 16, 16)
                t = tok_v[sl]
                i0_v[sl] = jnp.clip(t - _CUT[0], 0, _CUT[1] - _CUT[0] - 1)
                i1_v[sl] = jnp.clip(t - _CUT[1], 0, _CUT[2] - _CUT[1] - 1)
                i2_v[sl] = jnp.clip(t - _CUT[2], 0, _CUT[3] - _CUT[2] - 1) >> 1
                i3_v[sl] = jnp.clip(t - _CUT[3], 0, _CUT[4] - _CUT[3] - 1) >> 3
        tabs = (e0_h, e1_h, e2_h, e3_h)
        outs = (g0_h, g1_h, g2_h, g3_h)
        bufs = (b0_v, b1_v, b2_v, b3_v)
        idxs = (i0_v, i1_v, i2_v, i3_v)
        for c in range(4):
            ch = _CHUNKS[c]
            with jax.named_scope(f"sc_gather_c{c}"):
                for j in range(_BPW // ch):
                    isl = pl.ds(j * ch, ch)
                    pltpu.async_copy(
                        tabs[c].at[idxs[c].at[isl]], bufs[c], sem).wait()
                    pltpu.sync_copy(
                        bufs[c], outs[c].at[pl.ds(base + j * ch, ch)])

    return body(inp_flat, emb0, emb1, emb2g, emb3g)


def _tc_project(inp3, g0, g1, g2, g3, p0, p1, p2, p3):
    blk = 256
    nb = _NTOK // blk

    def body(i_ref, g0_ref, g1_ref, g2_ref, g3_ref,
             p0_ref, p1_ref, p2_ref, p3_ref, o_ref):
        tok = i_ref[0]

        m0 = (tok >= _CUT[0]) & (tok < _CUT[1])
        x0 = jnp.where(m0, g0_ref[...], 0.0).astype(jnp.bfloat16)

        m1 = (tok >= _CUT[1]) & (tok < _CUT[2])
        x1 = jnp.where(m1, g1_ref[...], 0.0).astype(jnp.bfloat16)

        m2 = (tok >= _CUT[2]) & (tok < _CUT[3])
        l2 = jnp.clip(tok - _CUT[2], 0, _CUT[3] - _CUT[2] - 1)
        half = (l2 & 1) == 1
        gr2 = g2_ref[...]
        x2f = jnp.where(half, gr2[:, 64:], gr2[:, :64])
        x2 = jnp.where(m2, x2f, 0.0).astype(jnp.bfloat16)

        m3 = (tok >= _CUT[3]) & (tok < _CUT[4])
        l3 = jnp.clip(tok - _CUT[3], 0, _CUT[4] - _CUT[3] - 1)
        slot = l3 & 7
        gr3 = g3_ref[...]
        x3f = jnp.zeros((blk, 16), jnp.float32)
        for s in range(8):
            x3f = jnp.where(slot == s,
                            gr3[:, s * 16:(s + 1) * 16], x3f)
        x3 = jnp.where(m3, x3f, 0.0).astype(jnp.bfloat16)

        dn = (((1,), (1,)), ((), ()))
        acc = lax.dot_general(x0, p0_ref[...], dn,
                              preferred_element_type=jnp.float32)
        acc += lax.dot_general(x1, p1_ref[...], dn,
                               preferred_element_type=jnp.float32)
        acc += lax.dot_general(x2, p2_ref[...], dn,
                               preferred_element_type=jnp.float32)
        acc += lax.dot_general(x3, p3_ref[...], dn,
                               preferred_element_type=jnp.float32)
        o_ref[...] = acc * _SCALE

    in_specs = [
        pl.BlockSpec((1, blk, 1), lambda i: (i, 0, 0)),
        pl.BlockSpec((blk, _GW[0]), lambda i: (i, 0)),
        pl.BlockSpec((blk, _GW[1]), lambda i: (i, 0)),
        pl.BlockSpec((blk, _GW[2]), lambda i: (i, 0)),
        pl.BlockSpec((blk, _GW[3]), lambda i: (i, 0)),
        pl.BlockSpec((_DPROJ, _DS[0]), lambda i: (0, 0)),
        pl.BlockSpec((_DPROJ, _DS[1]), lambda i: (0, 0)),
        pl.BlockSpec((_DPROJ, _DS[2]), lambda i: (0, 0)),
        pl.BlockSpec((_DPROJ, _DS[3]), lambda i: (0, 0)),
    ]
    return pl.pallas_call(
        body,
        grid=(nb,),
        in_specs=in_specs,
        out_specs=pl.BlockSpec((blk, _DPROJ), lambda i: (i, 0)),
        out_shape=jax.ShapeDtypeStruct((_NTOK, _DPROJ), jnp.float32),
        compiler_params=pltpu.CompilerParams(
            dimension_semantics=("arbitrary",)),
    )(inp3, g0, g1, g2, g3, p0, p1, p2, p3)


def kernel(inp, emb0, emb1, emb2, emb3, proj0, proj1, proj2, proj3):
    inp_flat = inp.reshape(-1)
    emb2g = emb2.reshape(-1, 128)
    emb3p = jnp.concatenate(
        [emb3, jnp.zeros((5, 16), jnp.float32)], axis=0)
    emb3g = emb3p.reshape(-1, 128)
    g0, g1, g2, g3 = _sc_gather(inp_flat, emb0, emb1, emb2g, emb3g)
    inp3 = inp_flat.reshape(_NTOK // 256, 256, 1)
    pbf = [p.astype(jnp.bfloat16) for p in (proj0, proj1, proj2, proj3)]
    out = _tc_project(inp3, g0, g1, g2, g3, *pbf)
    return out.reshape(inp.shape + (_DPROJ,))


# speedup vs baseline: 2.2425x; 2.2407x over previous
import functools

import jax
import jax.numpy as jnp
from jax import lax
from jax.experimental import pallas as pl
from jax.experimental.pallas import tpu as pltpu
from jax.experimental.pallas import tpu_sc as plsc

_NTOK = 8192
_NW = 32
_BPW = _NTOK // _NW
_CUT = (0, 19997, 39997, 199997, 1000000)
_GW = (1024, 256, 128, 128)
_DPROJ = 1024
_SCALE = float(_DPROJ) ** 0.5
_CAP = _BPW + 16


def _sc_gather(inp_flat, emb0, emb1, emb2g, emb3g):
    mesh = plsc.VectorSubcoreMesh(core_axis_name="c", subcore_axis_name="s")
    out_type = tuple(
        jax.ShapeDtypeStruct((_NTOK + (16 if c < 2 else 0), w), jnp.float32)
        for c, w in enumerate(_GW)
    )
    scratch = [
        pltpu.VMEM((_BPW,), jnp.int32),
        pltpu.VMEM((_BPW,), jnp.int32),
        pltpu.VMEM((_BPW,), jnp.int32),
        pltpu.VMEM((_CAP,), jnp.int32),
        pltpu.VMEM((_CAP,), jnp.int32),
        pltpu.VMEM((_CAP,), jnp.int32),
        pltpu.VMEM((_CAP,), jnp.int32),
        pltpu.VMEM((16, 1024), jnp.float32),
        pltpu.VMEM((16, 256), jnp.float32),
        pltpu.VMEM((128, 128), jnp.float32),
        pltpu.VMEM((128, 128), jnp.float32),
        pltpu.VMEM((128, 128), jnp.float32),
        pltpu.VMEM((128, 128), jnp.float32),
        pltpu.SemaphoreType.DMA,
        pltpu.SemaphoreType.DMA,
        pltpu.SemaphoreType.DMA,
        pltpu.SemaphoreType.DMA,
        pltpu.SemaphoreType.DMA,
    ]

    @functools.partial(pl.kernel, mesh=mesh, out_type=out_type,
                       scratch_types=scratch,
                       compiler_params=pltpu.CompilerParams(
                           needs_layout_passes=False))
    def body(inp_h, e0_h, e1_h, e2_h, e3_h, g0_h, g1_h, g2_h, g3_h,
             tok_v, i2_v, i3_v, li0_v, po0_v, li1_v, po1_v,
             b0_v, b1_v, b2a_v, b2b_v, b3a_v, b3b_v,
             sem0, sem2a, sem2b, sem3a, sem3b):
        wid = lax.axis_index("s") * 2 + lax.axis_index("c")
        base = wid * _BPW
        pltpu.sync_copy(inp_h.at[pl.ds(base, _BPW)], tok_v)

        for g in range(_BPW // 16):
            sl = pl.ds(g * 16, 16)
            t = tok_v[sl]
            i2_v[sl] = jnp.clip(t - _CUT[2], 0, _CUT[3] - _CUT[2] - 1) >> 1
            i3_v[sl] = jnp.minimum(
                jnp.clip(t - _CUT[3], 0, _CUT[4] - _CUT[3] - 1), 799999) >> 3

        _narrow = True
        if _narrow:
            cp2a = pltpu.async_copy(
                e2_h.at[i2_v.at[pl.ds(0, 128)]], b2a_v, sem2a)
            cp2b = pltpu.async_copy(
                e2_h.at[i2_v.at[pl.ds(128, 128)]], b2b_v, sem2b)
            cp3a = pltpu.async_copy(
                e3_h.at[i3_v.at[pl.ds(0, 128)]], b3a_v, sem3a)
            cp3b = pltpu.async_copy(
                e3_h.at[i3_v.at[pl.ds(128, 128)]], b3b_v, sem3b)

        zero = jnp.zeros((16,), jnp.int32)
        lane = lax.iota(jnp.int32, 16)
        trash = lane + _NTOK
        for g in range(_CAP // 16):
            sl = pl.ds(g * 16, 16)
            li0_v[sl] = zero
            po0_v[sl] = trash
            li1_v[sl] = zero
            po1_v[sl] = trash
        n0 = jnp.int32(0)
        n1 = jnp.int32(0)
        for g in range(_BPW // 16):
            t = tok_v[pl.ds(g * 16, 16)]
            pos = lane + (base + g * 16)
            m0 = t < _CUT[1]
            tgt0 = plsc.cumsum(m0.astype(jnp.int32)) + (n0 - 1)
            plsc.store_scatter(li0_v, [tgt0], t, mask=m0)
            plsc.store_scatter(po0_v, [tgt0], pos, mask=m0)
            n0 = n0 + jnp.sum(m0.astype(jnp.int32))
            m1 = (t >= _CUT[1]) & (t < _CUT[2])
            tgt1 = plsc.cumsum(m1.astype(jnp.int32)) + (n1 - 1)
            plsc.store_scatter(li1_v, [tgt1], t - _CUT[1], mask=m1)
            plsc.store_scatter(po1_v, [tgt1], pos, mask=m1)
            n1 = n1 + jnp.sum(m1.astype(jnp.int32))

        def c0_body(i, carry):
            iv = li0_v[pl.ds(i * 16, 16)]
            pltpu.async_copy(e0_h.at[iv], b0_v, sem0).wait()
            dv = po0_v[pl.ds(i * 16, 16)]
            pltpu.async_copy(b0_v, g0_h.at[dv], sem0).wait()
            return carry
        lax.fori_loop(0, (n0 + 15) // 16, c0_body, 0)

        def c1_body(i, carry):
            iv = li1_v[pl.ds(i * 16, 16)]
            pltpu.async_copy(e1_h.at[iv], b1_v, sem0).wait()
            dv = po1_v[pl.ds(i * 16, 16)]
            pltpu.async_copy(b1_v, g1_h.at[dv], sem0).wait()
            return carry
        lax.fori_loop(0, (n1 + 15) // 16, c1_body, 0)

        if _narrow:
            cp2a.wait()
            pltpu.sync_copy(b2a_v, g2_h.at[pl.ds(base, 128)])
            cp2b.wait()
            pltpu.sync_copy(b2b_v, g2_h.at[pl.ds(base + 128, 128)])
            cp3a.wait()
            pltpu.sync_copy(b3a_v, g3_h.at[pl.ds(base, 128)])
            cp3b.wait()
            pltpu.sync_copy(b3b_v, g3_h.at[pl.ds(base + 128, 128)])

    return body(inp_flat, emb0, emb1, emb2g, emb3g)


def _tc_project(inp3, g0, g1, g2, g3, tail, p0, p1, p2, p3):
    blk = 256
    nb = _NTOK // blk

    def body(i_ref, g0_ref, g1_ref, g2_ref, g3_ref, t_ref,
             p0_ref, p1_ref, p2_ref, p3_ref, o_ref):
        tok = i_ref[0]

        m0 = tok < _CUT[1]
        x0 = jnp.where(m0, g0_ref[...], 0.0).astype(jnp.bfloat16)

        m1 = (tok >= _CUT[1]) & (tok < _CUT[2])
        x1 = jnp.where(m1, g1_ref[...], 0.0).astype(jnp.bfloat16)

        m2 = (tok >= _CUT[2]) & (tok < _CUT[3])
        l2 = jnp.clip(tok - _CUT[2], 0, _CUT[3] - _CUT[2] - 1)
        half = (l2 & 1) == 1
        gr2 = g2_ref[...]
        x2f = jnp.where(half, gr2[:, 64:], gr2[:, :64])
        x2 = jnp.where(m2, x2f, 0.0).astype(jnp.bfloat16)

        m3 = tok >= _CUT[3]
        l3 = jnp.clip(tok - _CUT[3], 0, _CUT[4] - _CUT[3] - 1)
        slot = l3 & 7
        gr3 = g3_ref[...]
        x3f = jnp.zeros((blk, 16), jnp.float32)
        for s in range(8):
            x3f = jnp.where(slot == s,
                            gr3[:, s * 16:(s + 1) * 16], x3f)
        for s in range(3):
            x3f = jnp.where(l3 == 800000 + s, t_ref[s:s + 1, :], x3f)
        x3 = jnp.where(m3, x3f, 0.0).astype(jnp.bfloat16)

        dn = (((1,), (1,)), ((), ()))
        acc = lax.dot_general(x0, p0_ref[...], dn,
                              preferred_element_type=jnp.float32)
        acc += lax.dot_general(x1, p1_ref[...], dn,
                               preferred_element_type=jnp.float32)
        acc += lax.dot_general(x2, p2_ref[...], dn,
                               preferred_element_type=jnp.float32)
        acc += lax.dot_general(x3, p3_ref[...], dn,
                               preferred_element_type=jnp.float32)
        o_ref[...] = acc * _SCALE

    in_specs = [
        pl.BlockSpec((1, blk, 1), lambda i: (i, 0, 0)),
        pl.BlockSpec((blk, _GW[0]), lambda i: (i, 0)),
        pl.BlockSpec((blk, _GW[1]), lambda i: (i, 0)),
        pl.BlockSpec((blk, _GW[2]), lambda i: (i, 0)),
        pl.BlockSpec((blk, _GW[3]), lambda i: (i, 0)),
        pl.BlockSpec((8, 16), lambda i: (0, 0)),
        pl.BlockSpec((_DPROJ, 1024), lambda i: (0, 0)),
        pl.BlockSpec((_DPROJ, 256), lambda i: (0, 0)),
        pl.BlockSpec((_DPROJ, 64), lambda i: (0, 0)),
        pl.BlockSpec((_DPROJ, 16), lambda i: (0, 0)),
    ]
    return pl.pallas_call(
        body,
        grid=(nb,),
        in_specs=in_specs,
        out_specs=pl.BlockSpec((blk, _DPROJ), lambda i: (i, 0)),
        out_shape=jax.ShapeDtypeStruct((_NTOK, _DPROJ), jnp.float32),
        compiler_params=pltpu.CompilerParams(
            dimension_semantics=("arbitrary",)),
    )(inp3, g0, g1, g2, g3, tail, p0, p1, p2, p3)


def kernel(inp, emb0, emb1, emb2, emb3, proj0, proj1, proj2, proj3):
    inp_flat = inp.reshape(-1)
    emb2g = emb2.reshape(-1, 128)
    emb3g = emb3[:800000].reshape(-1, 128)
    tail = jnp.pad(emb3[800000:], ((0, 5), (0, 0)))
    g0, g1, g2, g3 = _sc_gather(inp_flat, emb0, emb1, emb2g, emb3g)
    inp3 = inp_flat.reshape(_NTOK // 256, 256, 1)
    pbf = [p.astype(jnp.bfloat16) for p in (proj0, proj1, proj2, proj3)]
    out = _tc_project(inp3, g0, g1, g2, g3, tail, *pbf)
    return out.reshape(inp.shape + (_DPROJ,))


# speedup vs baseline: 2.2516x; 1.0041x over previous
import functools

import jax
import jax.numpy as jnp
from jax import lax
from jax.experimental import pallas as pl
from jax.experimental.pallas import tpu as pltpu
from jax.experimental.pallas import tpu_sc as plsc

_NTOK = 8192
_NW = 32
_BPW = _NTOK // _NW
_CUT = (0, 19997, 39997, 199997, 1000000)
_GW = (1024, 256, 128, 128)
_DPROJ = 1024
_SCALE = float(_DPROJ) ** 0.5
_CAP = _BPW + 16


def _sc_gather(inp_flat, emb0, emb1, emb2g, emb3g):
    mesh = plsc.VectorSubcoreMesh(core_axis_name="c", subcore_axis_name="s")
    out_type = tuple(
        jax.ShapeDtypeStruct((_NTOK + (16 if c < 2 else 0), w), jnp.float32)
        for c, w in enumerate(_GW)
    )
    scratch = [
        pltpu.VMEM((_BPW,), jnp.int32),
        pltpu.VMEM((_BPW,), jnp.int32),
        pltpu.VMEM((_BPW,), jnp.int32),
        pltpu.VMEM((_CAP,), jnp.int32),
        pltpu.VMEM((_CAP,), jnp.int32),
        pltpu.VMEM((_CAP,), jnp.int32),
        pltpu.VMEM((_CAP,), jnp.int32),
        pltpu.VMEM((16, 1024), jnp.float32),
        pltpu.VMEM((16, 256), jnp.float32),
        pltpu.VMEM((128, 128), jnp.float32),
        pltpu.VMEM((128, 128), jnp.float32),
        pltpu.VMEM((128, 128), jnp.float32),
        pltpu.VMEM((128, 128), jnp.float32),
        pltpu.SemaphoreType.DMA,
        pltpu.SemaphoreType.DMA,
        pltpu.SemaphoreType.DMA,
        pltpu.SemaphoreType.DMA,
        pltpu.SemaphoreType.DMA,
        pltpu.SemaphoreType.DMA,
    ]

    @functools.partial(pl.kernel, mesh=mesh, out_type=out_type,
                       scratch_types=scratch,
                       compiler_params=pltpu.CompilerParams(
                           needs_layout_passes=False))
    def body(inp_h, e0_h, e1_h, e2_h, e3_h, g0_h, g1_h, g2_h, g3_h,
             tok_v, i2_v, i3_v, li0_v, po0_v, li1_v, po1_v,
             b0_v, b1_v, b2a_v, b2b_v, b3a_v, b3b_v,
             sem0, sem1, sem2a, sem2b, sem3a, sem3b):
        wid = lax.axis_index("s") * 2 + lax.axis_index("c")
        base = wid * _BPW
        pltpu.sync_copy(inp_h.at[pl.ds(base, _BPW)], tok_v)

        for g in range(_BPW // 16):
            sl = pl.ds(g * 16, 16)
            t = tok_v[sl]
            i2_v[sl] = jnp.clip(t - _CUT[2], 0, _CUT[3] - _CUT[2] - 1) >> 1
            i3_v[sl] = jnp.minimum(
                jnp.clip(t - _CUT[3], 0, _CUT[4] - _CUT[3] - 1), 799999) >> 3

        cp2a = pltpu.async_copy(
            e2_h.at[i2_v.at[pl.ds(0, 128)]], b2a_v, sem2a)
        cp2b = pltpu.async_copy(
            e2_h.at[i2_v.at[pl.ds(128, 128)]], b2b_v, sem2b)
        cp3a = pltpu.async_copy(
            e3_h.at[i3_v.at[pl.ds(0, 128)]], b3a_v, sem3a)
        cp3b = pltpu.async_copy(
            e3_h.at[i3_v.at[pl.ds(128, 128)]], b3b_v, sem3b)

        zero = jnp.zeros((16,), jnp.int32)
        lane = lax.iota(jnp.int32, 16)
        trash = lane + _NTOK
        for g in range(_CAP // 16):
            sl = pl.ds(g * 16, 16)
            li0_v[sl] = zero
            po0_v[sl] = trash
            li1_v[sl] = zero
            po1_v[sl] = trash
        n0 = jnp.int32(0)
        n1 = jnp.int32(0)
        for g in range(_BPW // 16):
            t = tok_v[pl.ds(g * 16, 16)]
            pos = lane + (base + g * 16)
            m0 = t < _CUT[1]
            tgt0 = plsc.cumsum(m0.astype(jnp.int32)) + (n0 - 1)
            plsc.store_scatter(li0_v, [tgt0], t, mask=m0)
            plsc.store_scatter(po0_v, [tgt0], pos, mask=m0)
            n0 = n0 + jnp.sum(m0.astype(jnp.int32))
            m1 = (t >= _CUT[1]) & (t < _CUT[2])
            tgt1 = plsc.cumsum(m1.astype(jnp.int32)) + (n1 - 1)
            plsc.store_scatter(li1_v, [tgt1], t - _CUT[1], mask=m1)
            plsc.store_scatter(po1_v, [tgt1], pos, mask=m1)
            n1 = n1 + jnp.sum(m1.astype(jnp.int32))

        iv0 = li0_v[pl.ds(0, 16)]
        iv1 = li1_v[pl.ds(0, 16)]
        cg0 = pltpu.async_copy(e0_h.at[iv0], b0_v, sem0)
        cg1 = pltpu.async_copy(e1_h.at[iv1], b1_v, sem1)
        dv0 = po0_v[pl.ds(0, 16)]
        dv1 = po1_v[pl.ds(0, 16)]
        cg0.wait()
        cs0 = pltpu.async_copy(b0_v, g0_h.at[dv0], sem0)
        cg1.wait()
        cs1 = pltpu.async_copy(b1_v, g1_h.at[dv1], sem1)

        cp2a.wait()
        cw2a = pltpu.async_copy(b2a_v, g2_h.at[pl.ds(base, 128)], sem2a)
        cp2b.wait()
        cw2b = pltpu.async_copy(b2b_v, g2_h.at[pl.ds(base + 128, 128)],
                                sem2b)
        cp3a.wait()
        cw3a = pltpu.async_copy(b3a_v, g3_h.at[pl.ds(base, 128)], sem3a)
        cp3b.wait()
        cw3b = pltpu.async_copy(b3b_v, g3_h.at[pl.ds(base + 128, 128)],
                                sem3b)

        cs0.wait()
        cs1.wait()

        def c0_body(i, carry):
            iv = li0_v[pl.ds(i * 16, 16)]
            pltpu.async_copy(e0_h.at[iv], b0_v, sem0).wait()
            dv = po0_v[pl.ds(i * 16, 16)]
            pltpu.async_copy(b0_v, g0_h.at[dv], sem0).wait()
            return carry
        lax.fori_loop(1, (n0 + 15) // 16, c0_body, 0)

        def c1_body(i, carry):
            iv = li1_v[pl.ds(i * 16, 16)]
            pltpu.async_copy(e1_h.at[iv], b1_v, sem1).wait()
            dv = po1_v[pl.ds(i * 16, 16)]
            pltpu.async_copy(b1_v, g1_h.at[dv], sem1).wait()
            return carry
        lax.fori_loop(1, (n1 + 15) // 16, c1_body, 0)

        cw2a.wait()
        cw2b.wait()
        cw3a.wait()
        cw3b.wait()

    return body(inp_flat, emb0, emb1, emb2g, emb3g)


def _tc_project(inp3, g0, g1, g2, g3, tail, p0, p1, p2, p3):
    blk = 256
    nb = _NTOK // blk

    def body(i_ref, g0_ref, g1_ref, g2_ref, g3_ref, t_ref,
             p0_ref, p1_ref, p2_ref, p3_ref, o_ref):
        tok = i_ref[0]

        m0 = tok < _CUT[1]
        x0 = jnp.where(m0, g0_ref[...], 0.0).astype(jnp.bfloat16)

        m1 = (tok >= _CUT[1]) & (tok < _CUT[2])
        x1 = jnp.where(m1, g1_ref[...], 0.0).astype(jnp.bfloat16)

        m2 = (tok >= _CUT[2]) & (tok < _CUT[3])
        l2 = jnp.clip(tok - _CUT[2], 0, _CUT[3] - _CUT[2] - 1)
        half = (l2 & 1) == 1
        gr2 = g2_ref[...]
        x2f = jnp.where(half, gr2[:, 64:], gr2[:, :64])
        x2 = jnp.where(m2, x2f, 0.0).astype(jnp.bfloat16)

        m3 = tok >= _CUT[3]
        l3 = jnp.clip(tok - _CUT[3], 0, _CUT[4] - _CUT[3] - 1)
        slot = l3 & 7
        gr3 = g3_ref[...]
        x3f = jnp.zeros((blk, 16), jnp.float32)
        for s in range(8):
            x3f = jnp.where(slot == s,
                            gr3[:, s * 16:(s + 1) * 16], x3f)
        for s in range(3):
            x3f = jnp.where(l3 == 800000 + s, t_ref[s:s + 1, :], x3f)
        x3 = jnp.where(m3, x3f, 0.0).astype(jnp.bfloat16)

        dn = (((1,), (1,)), ((), ()))
        acc = lax.dot_general(x0, p0_ref[...], dn,
                              preferred_element_type=jnp.float32)
        acc += lax.dot_general(x1, p1_ref[...], dn,
                               preferred_element_type=jnp.float32)
        acc += lax.dot_general(x2, p2_ref[...], dn,
                               preferred_element_type=jnp.float32)
        acc += lax.dot_general(x3, p3_ref[...], dn,
                               preferred_element_type=jnp.float32)
        o_ref[...] = acc * _SCALE

    in_specs = [
        pl.BlockSpec((1, blk, 1), lambda i: (i, 0, 0)),
        pl.BlockSpec((blk, _GW[0]), lambda i: (i, 0)),
        pl.BlockSpec((blk, _GW[1]), lambda i: (i, 0)),
        pl.BlockSpec((blk, _GW[2]), lambda i: (i, 0)),
        pl.BlockSpec((blk, _GW[3]), lambda i: (i, 0)),
        pl.BlockSpec((8, 16), lambda i: (0, 0)),
        pl.BlockSpec((_DPROJ, 1024), lambda i: (0, 0)),
        pl.BlockSpec((_DPROJ, 256), lambda i: (0, 0)),
        pl.BlockSpec((_DPROJ, 64), lambda i: (0, 0)),
        pl.BlockSpec((_DPROJ, 16), lambda i: (0, 0)),
    ]
    return pl.pallas_call(
        body,
        grid=(nb,),
        in_specs=in_specs,
        out_specs=pl.BlockSpec((blk, _DPROJ), lambda i: (i, 0)),
        out_shape=jax.ShapeDtypeStruct((_NTOK, _DPROJ), jnp.float32),
        compiler_params=pltpu.CompilerParams(
            dimension_semantics=("arbitrary",)),
    )(inp3, g0, g1, g2, g3, tail, p0, p1, p2, p3)


def kernel(inp, emb0, emb1, emb2, emb3, proj0, proj1, proj2, proj3):
    inp_flat = inp.reshape(-1)
    emb2g = emb2.reshape(-1, 128)
    emb3g = emb3[:800000].reshape(-1, 128)
    tail = jnp.pad(emb3[800000:], ((0, 5), (0, 0)))
    g0, g1, g2, g3 = _sc_gather(inp_flat, emb0, emb1, emb2g, emb3g)
    inp3 = inp_flat.reshape(_NTOK // 256, 256, 1)
    pbf = [p.astype(jnp.bfloat16) for p in (proj0, proj1, proj2, proj3)]
    out = _tc_project(inp3, g0, g1, g2, g3, tail, *pbf)
    return out.reshape(inp.shape + (_DPROJ,))
